# Initial kernel scaffold; baseline (speedup 1.0000x reference)
import jax
import jax.numpy as jnp
from jax.experimental import pallas as pl


def kernel(h, edge_index, W_gat, attn_l, attn_r, b_gat, W_mp, b_mp, W_cheb, b_cheb, W_ffn, b_ffn, W_fl, b_fl):
    raise NotImplementedError("write your pallas kernel here")


# speedup vs baseline: 48.8086x; 48.8086x over previous
import functools

import jax
import jax.numpy as jnp
from jax import lax
from jax.experimental import pallas as pl
from jax.experimental.pallas import tpu as pltpu
from jax.experimental.pallas import tpu_sc as plsc

NN = 10000
EE = 320000
FD = 128
GD = 256
HH = 8
DD = 16
KK = 4

NC = 2
NS = 16
NW = NC * NS
EPT = EE // NW
C = 40
NCHUNK = EPT // C
NP_ = 10240
ROWS = NP_ // NS

f32 = jnp.float32
i32 = jnp.int32


def _mesh():
    return plsc.VectorSubcoreMesh(core_axis_name="c", subcore_axis_name="s")


R624 = 624
TAIL0 = 15 * R624 + R624


def _sc_gat(G, B, src, dst, m16, zz128):
    @functools.partial(
        pl.kernel,
        out_type=[
            jax.ShapeDtypeStruct((EE, 16), f32),
            jax.ShapeDtypeStruct((NC * NN, FD), f32),
        ],
        mesh=_mesh(),
        scratch_types=[
            pltpu.VMEM((C,), i32),
            pltpu.VMEM((C,), i32),
            pltpu.VMEM((C, GD), f32),
            pltpu.VMEM((C, FD), f32),
            pltpu.VMEM((C, FD), f32),
            pltpu.VMEM((C, 16), f32),
            pltpu.VMEM((16,), f32),
            pltpu.VMEM_SHARED((NN, FD), f32),
            pltpu.SemaphoreType.DMA,
            pltpu.SemaphoreType.DMA,
        ],
    )
    def k(G_h, B_h, src_h, dst_h, m_h, zz128_h, ee_h, hg_h,
          si, di, ga, bb, msg, eer, mv, accS, sem1, sem2):
        cid = lax.axis_index("c")
        sid = lax.axis_index("s")
        wid = sid * NC + cid
        pltpu.sync_copy(zz128_h.at[pl.ds(sid * R624, R624)],
                        accS.at[pl.ds(sid * R624, R624)])

        @pl.when(sid == NS - 1)
        def _():
            pltpu.sync_copy(zz128_h.at[pl.ds(TAIL0, NN - TAIL0)],
                            accS.at[pl.ds(TAIL0, NN - TAIL0)])

        pltpu.sync_copy(m_h, mv)
        plsc.subcore_barrier()
        mvec = mv[...]
        base0 = wid * EPT

        def chunk(i, carry):
            base = base0 + i * C
            pltpu.sync_copy(src_h.at[pl.ds(base, C)], si)
            pltpu.sync_copy(dst_h.at[pl.ds(base, C)], di)
            cp1 = pltpu.async_copy(G_h.at[si], ga, sem1)
            cp2 = pltpu.async_copy(B_h.at[di], bb, sem2)
            cp1.wait()
            cp2.wait()

            def inner(c, carry2):
                v = ga[c, pl.ds(FD, 16)] + bb[c, pl.ds(0, 16)]
                v = jnp.where(v > 0.0, v, 0.2 * v)
                v = jnp.exp(v - mvec)
                eer[c, :] = v
                for hh in range(HH):
                    sl = pl.ds(hh * DD, DD)
                    msg[c, sl] = ga[c, sl] * v[hh]
                return carry2

            lax.fori_loop(0, C, inner, 0)
            pltpu.sync_copy(eer, ee_h.at[pl.ds(base, C)])
            pltpu.sync_copy(msg, accS.at[di], add=True)
            return carry

        lax.fori_loop(0, NCHUNK, chunk, 0)
        plsc.subcore_barrier()
        pltpu.sync_copy(accS.at[pl.ds(sid * R624, R624)],
                        hg_h.at[pl.ds(cid * NN + sid * R624, R624)])

        @pl.when(sid == NS - 1)
        def _():
            pltpu.sync_copy(accS.at[pl.ds(TAIL0, NN - TAIL0)],
                            hg_h.at[pl.ds(cid * NN + TAIL0, NN - TAIL0)])

    return k(G, B, src, dst, m16, zz128)


def _sc_esum(ee, dst, zz128):
    @functools.partial(
        pl.kernel,
        out_type=jax.ShapeDtypeStruct((NC * NP_, FD), f32),
        mesh=_mesh(),
        scratch_types=[
            pltpu.VMEM((C,), i32),
            pltpu.VMEM((C, 16), f32),
            pltpu.VMEM((C, FD), f32),
            pltpu.VMEM_SHARED((NP_, FD), f32),
        ],
    )
    def k(ee_h, dst_h, zz128_h, acc_h, di, eev, wide, accS):
        cid = lax.axis_index("c")
        sid = lax.axis_index("s")
        wid = sid * NC + cid
        pltpu.sync_copy(zz128_h.at[pl.ds(sid * ROWS, ROWS)],
                        accS.at[pl.ds(sid * ROWS, ROWS)])
        pltpu.sync_copy(zz128_h.at[pl.ds(0, C)], wide)
        plsc.subcore_barrier()
        base0 = wid * EPT

        def chunk(i, carry):
            base = base0 + i * C
            pltpu.sync_copy(dst_h.at[pl.ds(base, C)], di)
            pltpu.sync_copy(ee_h.at[pl.ds(base, C)], eev)

            def inner(c, carry2):
                wide[c, pl.ds(0, 16)] = eev[c, :]
                return carry2

            lax.fori_loop(0, C, inner, 0, unroll=4)
            pltpu.sync_copy(wide, accS.at[di], add=True)
            return carry

        lax.fori_loop(0, NCHUNK, chunk, 0)
        plsc.subcore_barrier()
        pltpu.sync_copy(accS.at[pl.ds(sid * ROWS, ROWS)],
                        acc_h.at[pl.ds(cid * NP_ + sid * ROWS, ROWS)])

    return k(ee, dst, zz128)


def _sc_adjacency(x, src, dst, zz128):
    @functools.partial(
        pl.kernel,
        out_type=jax.ShapeDtypeStruct((NC * NP_, FD), f32),
        mesh=_mesh(),
        scratch_types=[
            pltpu.VMEM((C,), i32),
            pltpu.VMEM((C,), i32),
            pltpu.VMEM((C, FD), f32),
            pltpu.VMEM_SHARED((NP_, FD), f32),
            pltpu.SemaphoreType.DMA,
        ],
    )
    def k(x_h, src_h, dst_h, zz_h, out_h, si, di, rows, accS, sem):
        cid = lax.axis_index("c")
        sid = lax.axis_index("s")
        wid = sid * NC + cid
        pltpu.sync_copy(zz_h.at[pl.ds(sid * ROWS, ROWS)],
                        accS.at[pl.ds(sid * ROWS, ROWS)])
        plsc.subcore_barrier()
        base0 = wid * EPT

        def chunk(i, carry):
            base = base0 + i * C
            pltpu.sync_copy(src_h.at[pl.ds(base, C)], si)
            pltpu.sync_copy(dst_h.at[pl.ds(base, C)], di)
            pltpu.async_copy(x_h.at[si], rows, sem).wait()
            pltpu.sync_copy(rows, accS.at[di], add=True)
            return carry

        lax.fori_loop(0, NCHUNK, chunk, 0)
        plsc.subcore_barrier()
        pltpu.sync_copy(accS.at[pl.ds(sid * ROWS, ROWS)],
                        out_h.at[pl.ds(cid * NP_ + sid * ROWS, ROWS)])

    return k(x, src, dst, zz128)


_NB = 5
_BLK = NN // _NB


def _tc_proj(h, W_gat, AL, AR):
    def body(h_ref, w_ref, al_ref, ar_ref, g_ref, b_ref, el_ref, er_ref):
        f = jnp.dot(h_ref[...], w_ref[...], preferred_element_type=f32)
        el = jnp.dot(f, al_ref[...], preferred_element_type=f32)
        er = jnp.dot(f, ar_ref[...], preferred_element_type=f32)
        z = jnp.zeros((_BLK, GD - FD - HH), f32)
        g_ref[...] = jnp.concatenate([f, el, z], axis=1)
        b_ref[...] = jnp.concatenate([er, jnp.zeros((_BLK, FD - HH), f32)], axis=1)
        el_ref[...] = el
        er_ref[...] = er

    return pl.pallas_call(
        body,
        grid=(_NB,),
        in_specs=[
            pl.BlockSpec((_BLK, FD), lambda i: (i, 0)),
            pl.BlockSpec((FD, FD), lambda i: (0, 0)),
            pl.BlockSpec((FD, HH), lambda i: (0, 0)),
            pl.BlockSpec((FD, HH), lambda i: (0, 0)),
        ],
        out_specs=[
            pl.BlockSpec((_BLK, GD), lambda i: (i, 0)),
            pl.BlockSpec((_BLK, FD), lambda i: (i, 0)),
            pl.BlockSpec((_BLK, HH), lambda i: (i, 0)),
            pl.BlockSpec((_BLK, HH), lambda i: (i, 0)),
        ],
        out_shape=[
            jax.ShapeDtypeStruct((NN, GD), f32),
            jax.ShapeDtypeStruct((NN, FD), f32),
            jax.ShapeDtypeStruct((NN, HH), f32),
            jax.ShapeDtypeStruct((NN, HH), f32),
        ],
    )(h, W_gat, AL, AR)


def _tc_maxm(el, er):
    def body(el_ref, er_ref, m_ref):
        m_ref[...] = jnp.full(
            (1, 1),
            jnp.maximum(jnp.max(el_ref[...]) + jnp.max(er_ref[...]), 0.0), f32)

    return pl.pallas_call(
        body,
        out_shape=jax.ShapeDtypeStruct((1, 1), f32),
    )(el, er)


def _tc_denom(acc2, W_mp, bmp32, WF32, bffn32, R832):
    def body(acc_ref, wmp_ref, bmp_ref, wf_ref, bffn_ref, r_ref,
             invd_ref, rs_ref, pooled_ref):
        acc = acc_ref[0] + acc_ref[1]
        esum = acc[:, :HH]
        invd = 1.0 / (esum + 1e-16)
        invd_ref[...] = invd
        cnt = acc[:, HH:HH + 1]
        deg = jnp.maximum(cnt, 1.0)
        rs_ref[...] = lax.rsqrt(deg)
        s = esum * invd
        s32 = jnp.dot(s, r_ref[...], preferred_element_type=f32)
        colsum = jnp.sum(wmp_ref[...], axis=0, keepdims=True)
        cs32 = jnp.concatenate([colsum] * HH, axis=1)
        xc = jnp.tanh(s32 * cs32 + bmp_ref[...])
        hg = jnp.mean(xc, axis=0, keepdims=True)
        pooled_ref[...] = (
            jnp.dot(hg, wf_ref[...], preferred_element_type=f32) + bffn_ref[...])

    return pl.pallas_call(
        body,
        out_shape=[
            jax.ShapeDtypeStruct((NN, HH), f32),
            jax.ShapeDtypeStruct((NN, 1), f32),
            jax.ShapeDtypeStruct((1, HH * KK), f32),
        ],
    )(acc2, W_mp, bmp32, WF32, bffn32, R832)


def _tc_gatout(hg2, invd, rs, E8, bg128):
    def body(hg_ref, invd_ref, rs_ref, e8_ref, bg_ref, hgat_ref, y_ref):
        invd128 = jnp.dot(invd_ref[...], e8_ref[...], preferred_element_type=f32)
        hgat = (hg_ref[0] + hg_ref[1]) * invd128 + bg_ref[...]
        hgat_ref[...] = hgat
        y_ref[...] = rs_ref[...] * hgat

    return pl.pallas_call(
        body,
        grid=(_NB,),
        in_specs=[
            pl.BlockSpec((2, _BLK, FD), lambda i: (0, i, 0)),
            pl.BlockSpec((_BLK, HH), lambda i: (i, 0)),
            pl.BlockSpec((_BLK, 1), lambda i: (i, 0)),
            pl.BlockSpec((HH, FD), lambda i: (0, 0)),
            pl.BlockSpec((1, FD), lambda i: (0, 0)),
        ],
        out_specs=[
            pl.BlockSpec((_BLK, FD), lambda i: (i, 0)),
            pl.BlockSpec((_BLK, FD), lambda i: (i, 0)),
        ],
        out_shape=[
            jax.ShapeDtypeStruct((NN, FD), f32),
            jax.ShapeDtypeStruct((NN, FD), f32),
        ],
    )(hg2, invd, rs, E8, bg128)


def _tc_cheb_step(p2, rs, prev):
    first = prev is None
    coef = -1.0 if first else -2.0

    def body(*refs):
        if first:
            p_ref, rs_ref, tx_ref, y_ref = refs
            tx = coef * rs_ref[...] * (p_ref[0] + p_ref[1])
        else:
            p_ref, rs_ref, prev_ref, tx_ref, y_ref = refs
            tx = coef * rs_ref[...] * (p_ref[0] + p_ref[1]) - prev_ref[...]
        tx_ref[...] = tx
        y_ref[...] = rs_ref[...] * tx

    in_specs = [
        pl.BlockSpec((2, _BLK, FD), lambda i: (0, i, 0)),
        pl.BlockSpec((_BLK, 1), lambda i: (i, 0)),
    ]
    args = [p2, rs]
    if not first:
        in_specs.append(pl.BlockSpec((_BLK, FD), lambda i: (i, 0)))
        args.append(prev)
    return pl.pallas_call(
        body,
        grid=(_NB,),
        in_specs=in_specs,
        out_specs=[
            pl.BlockSpec((_BLK, FD), lambda i: (i, 0)),
            pl.BlockSpec((_BLK, FD), lambda i: (i, 0)),
        ],
        out_shape=[
            jax.ShapeDtypeStruct((NN, FD), f32),
            jax.ShapeDtypeStruct((NN, FD), f32),
        ],
    )(*args)


def _tc_final(p2, rs, tx1, tx2, hgat, pooled_exp, BD, BDfl, bc128, bfl128, h_in):
    def body(p_ref, rs_ref, tx1_ref, tx2_ref, hgat_ref, pe_ref, bd_ref,
             bdfl_ref, bc_ref, bfl_ref, hin_ref, out_ref):
        tx3 = -2.0 * rs_ref[...] * (p_ref[0] + p_ref[1]) - tx1_ref[...]
        txs = (hgat_ref[...], tx1_ref[...], tx2_ref[...], tx3)
        acc = jnp.zeros((_BLK, FD), f32) + bc_ref[...]
        for k in range(KK):
            acc = acc + jnp.dot(pe_ref[k:k + 1, :] * txs[k], bd_ref[k],
                                preferred_element_type=f32)
        hf = jnp.dot(jnp.tanh(acc), bdfl_ref[...],
                     preferred_element_type=f32) + bfl_ref[...]
        hh = hgat_ref[...] + hf
        hh = jnp.where(hh > 0.0, hh, jnp.exp(jnp.minimum(hh, 0.0)) - 1.0)
        out_ref[...] = hin_ref[...] + hh

    return pl.pallas_call(
        body,
        grid=(_NB,),
        in_specs=[
            pl.BlockSpec((2, _BLK, FD), lambda i: (0, i, 0)),
            pl.BlockSpec((_BLK, 1), lambda i: (i, 0)),
            pl.BlockSpec((_BLK, FD), lambda i: (i, 0)),
            pl.BlockSpec((_BLK, FD), lambda i: (i, 0)),
            pl.BlockSpec((_BLK, FD), lambda i: (i, 0)),
            pl.BlockSpec((KK, FD), lambda i: (0, 0)),
            pl.BlockSpec((KK, FD, FD), lambda i: (0, 0, 0)),
            pl.BlockSpec((FD, FD), lambda i: (0, 0)),
            pl.BlockSpec((1, FD), lambda i: (0, 0)),
            pl.BlockSpec((1, FD), lambda i: (0, 0)),
            pl.BlockSpec((_BLK, FD), lambda i: (i, 0)),
        ],
        out_specs=[pl.BlockSpec((_BLK, FD), lambda i: (i, 0))],
        out_shape=[jax.ShapeDtypeStruct((NN, FD), f32)],
    )(p2, rs, tx1, tx2, hgat, pooled_exp, BD, BDfl, bc128, bfl128, h_in)[0]


def kernel(h, edge_index, W_gat, attn_l, attn_r, b_gat, W_mp, b_mp,
           W_cheb, b_cheb, W_ffn, b_ffn, W_fl, b_fl):
    src = edge_index[0].astype(i32)
    dst = edge_index[1].astype(i32)

    lanes = jnp.arange(FD)
    rowh = lanes // DD
    AL = jnp.zeros((FD, HH), f32).at[lanes, rowh].set(attn_l.reshape(-1))
    AR = jnp.zeros((FD, HH), f32).at[lanes, rowh].set(attn_r.reshape(-1))
    E8 = jnp.zeros((HH, FD), f32).at[rowh, lanes].set(1.0)
    R832 = jnp.zeros((HH, HH * KK), f32).at[
        jnp.arange(HH * KK) // KK, jnp.arange(HH * KK)].set(1.0)
    bmp32 = jnp.tile(b_mp, HH).reshape(1, HH * KK)
    WF32 = jnp.kron(jnp.eye(HH, dtype=f32), W_ffn)
    bffn32 = jnp.tile(b_ffn, HH).reshape(1, HH * KK)
    eye8 = jnp.eye(HH, dtype=f32)
    BD = jnp.stack([jnp.kron(eye8, W_cheb[k]) for k in range(KK)])
    BDfl = jnp.kron(eye8, W_fl)
    bg128 = b_gat.reshape(1, FD)
    bc128 = jnp.tile(b_cheb, HH).reshape(1, FD)
    bfl128 = jnp.tile(b_fl, HH).reshape(1, FD)
    zz128 = jnp.zeros((NP_, FD), f32)
    zz128n = jnp.zeros((NN, FD), f32)

    G, B128, el, er = _tc_proj(h, W_gat, AL, AR)
    m1 = _tc_maxm(el, er)
    m16 = jnp.concatenate(
        [jnp.broadcast_to(m1.reshape(1), (HH,)), jnp.zeros((HH,), f32)])

    ee, hg = _sc_gat(G, B128, src, dst, m16, zz128n)
    acc = _sc_esum(ee, dst, zz128)

    invd, rs, pooled32 = _tc_denom(acc.reshape(NC, NP_, FD)[:, :NN, :16],
                                   W_mp, bmp32, WF32, bffn32, R832)
    pooled_exp = jnp.repeat(pooled32.reshape(HH, KK).T, DD, axis=1)

    hgat, y1 = _tc_gatout(hg.reshape(NC, NN, FD), invd, rs, E8, bg128)

    p1 = _sc_adjacency(y1, src, dst, zz128)
    tx1, y2 = _tc_cheb_step(p1.reshape(NC, NP_, FD)[:, :NN], rs, None)
    p2 = _sc_adjacency(y2, src, dst, zz128)
    tx2, y3 = _tc_cheb_step(p2.reshape(NC, NP_, FD)[:, :NN], rs, hgat)
    p3 = _sc_adjacency(y3, src, dst, zz128)

    return _tc_final(p3.reshape(NC, NP_, FD)[:, :NN], rs, tx1, tx2, hgat,
                     pooled_exp, BD, BDfl, bc128, bfl128, h)


# speedup vs baseline: 60.3509x; 1.2365x over previous
import functools

import jax
import jax.numpy as jnp
from jax import lax
from jax.experimental import pallas as pl
from jax.experimental.pallas import tpu as pltpu
from jax.experimental.pallas import tpu_sc as plsc

NN = 10000
EE = 320000
FD = 128
GD = 256
HH = 8
DD = 16
KK = 4

NC = 2
NS = 16
NW = NC * NS
EPT = EE // NW
C = 40
NCHUNK = EPT // C
NP_ = 10240
ROWS = NP_ // NS

f32 = jnp.float32
i32 = jnp.int32


def _mesh():
    return plsc.VectorSubcoreMesh(core_axis_name="c", subcore_axis_name="s")


R624 = 624
TAIL0 = 15 * R624 + R624


def _sc_gat(G, B, src, dst, m16, zz128):
    @functools.partial(
        pl.kernel,
        out_type=[
            jax.ShapeDtypeStruct((EE, 16), f32),
            jax.ShapeDtypeStruct((NC * NN, FD), f32),
        ],
        mesh=_mesh(),
        scratch_types=[
            pltpu.VMEM((C,), i32),
            pltpu.VMEM((C,), i32),
            pltpu.VMEM((C, GD), f32),
            pltpu.VMEM((C, FD), f32),
            pltpu.VMEM((C, FD), f32),
            pltpu.VMEM((C, 16), f32),
            pltpu.VMEM((16,), f32),
            pltpu.VMEM_SHARED((NN, FD), f32),
            pltpu.SemaphoreType.DMA,
            pltpu.SemaphoreType.DMA,
        ],
    )
    def k(G_h, B_h, src_h, dst_h, m_h, zz128_h, ee_h, hg_h,
          si, di, ga, bb, msg, eer, mv, accS, sem1, sem2):
        cid = lax.axis_index("c")
        sid = lax.axis_index("s")
        wid = sid * NC + cid
        pltpu.sync_copy(zz128_h.at[pl.ds(sid * R624, R624)],
                        accS.at[pl.ds(sid * R624, R624)])

        @pl.when(sid == NS - 1)
        def _():
            pltpu.sync_copy(zz128_h.at[pl.ds(TAIL0, NN - TAIL0)],
                            accS.at[pl.ds(TAIL0, NN - TAIL0)])

        pltpu.sync_copy(m_h, mv)
        plsc.subcore_barrier()
        mvec = mv[...]
        base0 = wid * EPT

        def chunk(i, carry):
            base = base0 + i * C
            pltpu.sync_copy(src_h.at[pl.ds(base, C)], si)
            pltpu.sync_copy(dst_h.at[pl.ds(base, C)], di)
            cp1 = pltpu.async_copy(G_h.at[si], ga, sem1)
            cp2 = pltpu.async_copy(B_h.at[di], bb, sem2)
            cp1.wait()
            cp2.wait()

            def inner(c, carry2):
                v = ga[c, pl.ds(FD, 16)] + bb[c, pl.ds(0, 16)]
                v = jnp.where(v > 0.0, v, 0.2 * v)
                v = jnp.exp(v - mvec)
                eer[c, :] = v
                for hh in range(HH):
                    sl = pl.ds(hh * DD, DD)
                    msg[c, sl] = ga[c, sl] * v[hh]
                return carry2

            lax.fori_loop(0, C, inner, 0, unroll=2)
            ce = pltpu.async_copy(eer, ee_h.at[pl.ds(base, C)], sem1)
            cs = pltpu.async_copy(msg, accS.at[di], sem2, add=True)
            ce.wait()
            cs.wait()
            return carry

        lax.fori_loop(0, NCHUNK, chunk, 0)
        plsc.subcore_barrier()
        pltpu.sync_copy(accS.at[pl.ds(sid * R624, R624)],
                        hg_h.at[pl.ds(cid * NN + sid * R624, R624)])

        @pl.when(sid == NS - 1)
        def _():
            pltpu.sync_copy(accS.at[pl.ds(TAIL0, NN - TAIL0)],
                            hg_h.at[pl.ds(cid * NN + TAIL0, NN - TAIL0)])

    return k(G, B, src, dst, m16, zz128)


def _sc_esum(ee, dst, zz128):
    @functools.partial(
        pl.kernel,
        out_type=jax.ShapeDtypeStruct((NC * NN, FD), f32),
        mesh=_mesh(),
        scratch_types=[
            pltpu.VMEM((C,), i32),
            pltpu.VMEM((C,), i32),
            pltpu.VMEM((C, 16), f32),
            pltpu.VMEM((C, 16), f32),
            pltpu.VMEM((C, FD), f32),
            pltpu.VMEM((C, FD), f32),
            pltpu.VMEM_SHARED((NN, FD), f32),
            pltpu.SemaphoreType.DMA,
            pltpu.SemaphoreType.DMA,
            pltpu.SemaphoreType.DMA,
            pltpu.SemaphoreType.DMA,
        ],
    )
    def k(ee_h, dst_h, zz128_h, acc_h, di0, di1, ev0, ev1, w0, w1, accS,
          g0, g1, s0, s1):
        cid = lax.axis_index("c")
        sid = lax.axis_index("s")
        wid = sid * NC + cid
        pltpu.sync_copy(zz128_h.at[pl.ds(sid * R624, R624)],
                        accS.at[pl.ds(sid * R624, R624)])

        @pl.when(sid == NS - 1)
        def _():
            pltpu.sync_copy(zz128_h.at[pl.ds(TAIL0, NN - TAIL0)],
                            accS.at[pl.ds(TAIL0, NN - TAIL0)])

        pltpu.sync_copy(zz128_h.at[pl.ds(0, C)], w0)
        pltpu.sync_copy(zz128_h.at[pl.ds(0, C)], w1)
        plsc.subcore_barrier()
        base0 = wid * EPT

        def expand(ev, w):
            def inner(c, carry2):
                w[c, pl.ds(0, 16)] = ev[c, :]
                return carry2

            lax.fori_loop(0, C, inner, 0, unroll=8)

        def pair(j, carry):
            a0 = base0 + (2 * j) * C
            a1 = a0 + C
            pltpu.sync_copy(dst_h.at[pl.ds(a0, C)], di0)
            cg0 = pltpu.async_copy(ee_h.at[pl.ds(a0, C)], ev0, g0)
            pltpu.sync_copy(dst_h.at[pl.ds(a1, C)], di1)
            cg1 = pltpu.async_copy(ee_h.at[pl.ds(a1, C)], ev1, g1)
            cg0.wait()
            expand(ev0, w0)
            cs0 = pltpu.async_copy(w0, accS.at[di0], s0, add=True)
            cg1.wait()
            expand(ev1, w1)
            cs1 = pltpu.async_copy(w1, accS.at[di1], s1, add=True)
            cs0.wait()
            cs1.wait()
            return carry

        lax.fori_loop(0, NCHUNK // 2, pair, 0)
        plsc.subcore_barrier()
        pltpu.sync_copy(accS.at[pl.ds(sid * R624, R624)],
                        acc_h.at[pl.ds(cid * NN + sid * R624, R624)])

        @pl.when(sid == NS - 1)
        def _():
            pltpu.sync_copy(accS.at[pl.ds(TAIL0, NN - TAIL0)],
                            acc_h.at[pl.ds(cid * NN + TAIL0, NN - TAIL0)])

    return k(ee, dst, zz128)


def _sc_adjacency(x, src, dst, zz128):
    @functools.partial(
        pl.kernel,
        out_type=jax.ShapeDtypeStruct((NC * NN, FD), f32),
        mesh=_mesh(),
        scratch_types=[
            pltpu.VMEM((C,), i32),
            pltpu.VMEM((C,), i32),
            pltpu.VMEM((C,), i32),
            pltpu.VMEM((C,), i32),
            pltpu.VMEM((C, FD), f32),
            pltpu.VMEM((C, FD), f32),
            pltpu.VMEM_SHARED((NN, FD), f32),
            pltpu.SemaphoreType.DMA,
            pltpu.SemaphoreType.DMA,
            pltpu.SemaphoreType.DMA,
            pltpu.SemaphoreType.DMA,
        ],
    )
    def k(x_h, src_h, dst_h, zz_h, out_h, si0, di0, si1, di1, r0, r1, accS,
          g0, g1, s0, s1):
        cid = lax.axis_index("c")
        sid = lax.axis_index("s")
        wid = sid * NC + cid
        pltpu.sync_copy(zz_h.at[pl.ds(sid * R624, R624)],
                        accS.at[pl.ds(sid * R624, R624)])

        @pl.when(sid == NS - 1)
        def _():
            pltpu.sync_copy(zz_h.at[pl.ds(TAIL0, NN - TAIL0)],
                            accS.at[pl.ds(TAIL0, NN - TAIL0)])

        plsc.subcore_barrier()
        base0 = wid * EPT

        def pair(j, carry):
            a0 = base0 + (2 * j) * C
            a1 = a0 + C
            pltpu.sync_copy(src_h.at[pl.ds(a0, C)], si0)
            pltpu.sync_copy(dst_h.at[pl.ds(a0, C)], di0)
            cg0 = pltpu.async_copy(x_h.at[si0], r0, g0)
            pltpu.sync_copy(src_h.at[pl.ds(a1, C)], si1)
            pltpu.sync_copy(dst_h.at[pl.ds(a1, C)], di1)
            cg1 = pltpu.async_copy(x_h.at[si1], r1, g1)
            cg0.wait()
            cs0 = pltpu.async_copy(r0, accS.at[di0], s0, add=True)
            cg1.wait()
            cs1 = pltpu.async_copy(r1, accS.at[di1], s1, add=True)
            cs0.wait()
            cs1.wait()
            return carry

        lax.fori_loop(0, NCHUNK // 2, pair, 0)
        plsc.subcore_barrier()
        pltpu.sync_copy(accS.at[pl.ds(sid * R624, R624)],
                        out_h.at[pl.ds(cid * NN + sid * R624, R624)])

        @pl.when(sid == NS - 1)
        def _():
            pltpu.sync_copy(accS.at[pl.ds(TAIL0, NN - TAIL0)],
                            out_h.at[pl.ds(cid * NN + TAIL0, NN - TAIL0)])

    return k(x, src, dst, zz128)


_NB = 5
_BLK = NN // _NB


def _tc_proj(h, W_gat, AL, AR):
    def body(h_ref, w_ref, al_ref, ar_ref, g_ref, b_ref, el_ref, er_ref):
        f = jnp.dot(h_ref[...], w_ref[...], preferred_element_type=f32)
        el = jnp.dot(f, al_ref[...], preferred_element_type=f32)
        er = jnp.dot(f, ar_ref[...], preferred_element_type=f32)
        z = jnp.zeros((_BLK, GD - FD - HH), f32)
        g_ref[...] = jnp.concatenate([f, el, z], axis=1)
        b_ref[...] = jnp.concatenate([er, jnp.zeros((_BLK, FD - HH), f32)], axis=1)
        el_ref[...] = el
        er_ref[...] = er

    return pl.pallas_call(
        body,
        grid=(_NB,),
        in_specs=[
            pl.BlockSpec((_BLK, FD), lambda i: (i, 0)),
            pl.BlockSpec((FD, FD), lambda i: (0, 0)),
            pl.BlockSpec((FD, HH), lambda i: (0, 0)),
            pl.BlockSpec((FD, HH), lambda i: (0, 0)),
        ],
        out_specs=[
            pl.BlockSpec((_BLK, GD), lambda i: (i, 0)),
            pl.BlockSpec((_BLK, FD), lambda i: (i, 0)),
            pl.BlockSpec((_BLK, HH), lambda i: (i, 0)),
            pl.BlockSpec((_BLK, HH), lambda i: (i, 0)),
        ],
        out_shape=[
            jax.ShapeDtypeStruct((NN, GD), f32),
            jax.ShapeDtypeStruct((NN, FD), f32),
            jax.ShapeDtypeStruct((NN, HH), f32),
            jax.ShapeDtypeStruct((NN, HH), f32),
        ],
    )(h, W_gat, AL, AR)


def _tc_maxm(el, er):
    def body(el_ref, er_ref, m_ref):
        m_ref[...] = jnp.full(
            (1, 1),
            jnp.maximum(jnp.max(el_ref[...]) + jnp.max(er_ref[...]), 0.0), f32)

    return pl.pallas_call(
        body,
        out_shape=jax.ShapeDtypeStruct((1, 1), f32),
    )(el, er)


def _tc_denom(acc2, W_mp, bmp32, WF32, bffn32, R832):
    def body(acc_ref, wmp_ref, bmp_ref, wf_ref, bffn_ref, r_ref,
             invd_ref, rs_ref, pooled_ref):
        acc = acc_ref[0] + acc_ref[1]
        esum = acc[:, :HH]
        invd = 1.0 / (esum + 1e-16)
        invd_ref[...] = invd
        cnt = acc[:, HH:HH + 1]
        deg = jnp.maximum(cnt, 1.0)
        rs_ref[...] = lax.rsqrt(deg)
        s = esum * invd
        s32 = jnp.dot(s, r_ref[...], preferred_element_type=f32)
        colsum = jnp.sum(wmp_ref[...], axis=0, keepdims=True)
        cs32 = jnp.concatenate([colsum] * HH, axis=1)
        xc = jnp.tanh(s32 * cs32 + bmp_ref[...])
        hg = jnp.mean(xc, axis=0, keepdims=True)
        pooled_ref[...] = (
            jnp.dot(hg, wf_ref[...], preferred_element_type=f32) + bffn_ref[...])

    return pl.pallas_call(
        body,
        out_shape=[
            jax.ShapeDtypeStruct((NN, HH), f32),
            jax.ShapeDtypeStruct((NN, 1), f32),
            jax.ShapeDtypeStruct((1, HH * KK), f32),
        ],
    )(acc2, W_mp, bmp32, WF32, bffn32, R832)


def _tc_gatout(hg2, invd, rs, E8, bg128):
    def body(hg_ref, invd_ref, rs_ref, e8_ref, bg_ref, hgat_ref, y_ref):
        invd128 = jnp.dot(invd_ref[...], e8_ref[...], preferred_element_type=f32)
        hgat = (hg_ref[0] + hg_ref[1]) * invd128 + bg_ref[...]
        hgat_ref[...] = hgat
        y_ref[...] = rs_ref[...] * hgat

    return pl.pallas_call(
        body,
        grid=(_NB,),
        in_specs=[
            pl.BlockSpec((2, _BLK, FD), lambda i: (0, i, 0)),
            pl.BlockSpec((_BLK, HH), lambda i: (i, 0)),
            pl.BlockSpec((_BLK, 1), lambda i: (i, 0)),
            pl.BlockSpec((HH, FD), lambda i: (0, 0)),
            pl.BlockSpec((1, FD), lambda i: (0, 0)),
        ],
        out_specs=[
            pl.BlockSpec((_BLK, FD), lambda i: (i, 0)),
            pl.BlockSpec((_BLK, FD), lambda i: (i, 0)),
        ],
        out_shape=[
            jax.ShapeDtypeStruct((NN, FD), f32),
            jax.ShapeDtypeStruct((NN, FD), f32),
        ],
    )(hg2, invd, rs, E8, bg128)


def _tc_cheb_step(p2, rs, prev):
    first = prev is None
    coef = -1.0 if first else -2.0

    def body(*refs):
        if first:
            p_ref, rs_ref, tx_ref, y_ref = refs
            tx = coef * rs_ref[...] * (p_ref[0] + p_ref[1])
        else:
            p_ref, rs_ref, prev_ref, tx_ref, y_ref = refs
            tx = coef * rs_ref[...] * (p_ref[0] + p_ref[1]) - prev_ref[...]
        tx_ref[...] = tx
        y_ref[...] = rs_ref[...] * tx

    in_specs = [
        pl.BlockSpec((2, _BLK, FD), lambda i: (0, i, 0)),
        pl.BlockSpec((_BLK, 1), lambda i: (i, 0)),
    ]
    args = [p2, rs]
    if not first:
        in_specs.append(pl.BlockSpec((_BLK, FD), lambda i: (i, 0)))
        args.append(prev)
    return pl.pallas_call(
        body,
        grid=(_NB,),
        in_specs=in_specs,
        out_specs=[
            pl.BlockSpec((_BLK, FD), lambda i: (i, 0)),
            pl.BlockSpec((_BLK, FD), lambda i: (i, 0)),
        ],
        out_shape=[
            jax.ShapeDtypeStruct((NN, FD), f32),
            jax.ShapeDtypeStruct((NN, FD), f32),
        ],
    )(*args)


def _tc_final(p2, rs, tx1, tx2, hgat, pooled_exp, BD, BDfl, bc128, bfl128, h_in):
    def body(p_ref, rs_ref, tx1_ref, tx2_ref, hgat_ref, pe_ref, bd_ref,
             bdfl_ref, bc_ref, bfl_ref, hin_ref, out_ref):
        tx3 = -2.0 * rs_ref[...] * (p_ref[0] + p_ref[1]) - tx1_ref[...]
        txs = (hgat_ref[...], tx1_ref[...], tx2_ref[...], tx3)
        acc = jnp.zeros((_BLK, FD), f32) + bc_ref[...]
        for k in range(KK):
            acc = acc + jnp.dot(pe_ref[k:k + 1, :] * txs[k], bd_ref[k],
                                preferred_element_type=f32)
        hf = jnp.dot(jnp.tanh(acc), bdfl_ref[...],
                     preferred_element_type=f32) + bfl_ref[...]
        hh = hgat_ref[...] + hf
        hh = jnp.where(hh > 0.0, hh, jnp.exp(jnp.minimum(hh, 0.0)) - 1.0)
        out_ref[...] = hin_ref[...] + hh

    return pl.pallas_call(
        body,
        grid=(_NB,),
        in_specs=[
            pl.BlockSpec((2, _BLK, FD), lambda i: (0, i, 0)),
            pl.BlockSpec((_BLK, 1), lambda i: (i, 0)),
            pl.BlockSpec((_BLK, FD), lambda i: (i, 0)),
            pl.BlockSpec((_BLK, FD), lambda i: (i, 0)),
            pl.BlockSpec((_BLK, FD), lambda i: (i, 0)),
            pl.BlockSpec((KK, FD), lambda i: (0, 0)),
            pl.BlockSpec((KK, FD, FD), lambda i: (0, 0, 0)),
            pl.BlockSpec((FD, FD), lambda i: (0, 0)),
            pl.BlockSpec((1, FD), lambda i: (0, 0)),
            pl.BlockSpec((1, FD), lambda i: (0, 0)),
            pl.BlockSpec((_BLK, FD), lambda i: (i, 0)),
        ],
        out_specs=[pl.BlockSpec((_BLK, FD), lambda i: (i, 0))],
        out_shape=[jax.ShapeDtypeStruct((NN, FD), f32)],
    )(p2, rs, tx1, tx2, hgat, pooled_exp, BD, BDfl, bc128, bfl128, h_in)[0]


def kernel(h, edge_index, W_gat, attn_l, attn_r, b_gat, W_mp, b_mp,
           W_cheb, b_cheb, W_ffn, b_ffn, W_fl, b_fl):
    src = edge_index[0].astype(i32)
    dst = edge_index[1].astype(i32)

    lanes = jnp.arange(FD)
    rowh = lanes // DD
    AL = jnp.zeros((FD, HH), f32).at[lanes, rowh].set(attn_l.reshape(-1))
    AR = jnp.zeros((FD, HH), f32).at[lanes, rowh].set(attn_r.reshape(-1))
    E8 = jnp.zeros((HH, FD), f32).at[rowh, lanes].set(1.0)
    R832 = jnp.zeros((HH, HH * KK), f32).at[
        jnp.arange(HH * KK) // KK, jnp.arange(HH * KK)].set(1.0)
    bmp32 = jnp.tile(b_mp, HH).reshape(1, HH * KK)
    WF32 = jnp.kron(jnp.eye(HH, dtype=f32), W_ffn)
    bffn32 = jnp.tile(b_ffn, HH).reshape(1, HH * KK)
    eye8 = jnp.eye(HH, dtype=f32)
    BD = jnp.stack([jnp.kron(eye8, W_cheb[k]) for k in range(KK)])
    BDfl = jnp.kron(eye8, W_fl)
    bg128 = b_gat.reshape(1, FD)
    bc128 = jnp.tile(b_cheb, HH).reshape(1, FD)
    bfl128 = jnp.tile(b_fl, HH).reshape(1, FD)
    zz128 = jnp.zeros((NP_, FD), f32)
    zz128n = jnp.zeros((NN, FD), f32)

    G, B128, el, er = _tc_proj(h, W_gat, AL, AR)
    m1 = _tc_maxm(el, er)
    m16 = jnp.concatenate(
        [jnp.broadcast_to(m1.reshape(1), (HH,)), jnp.zeros((HH,), f32)])

    ee, hg = _sc_gat(G, B128, src, dst, m16, zz128n)
    acc = _sc_esum(ee, dst, zz128)

    invd, rs, pooled32 = _tc_denom(acc.reshape(NC, NN, FD)[:, :, :16],
                                   W_mp, bmp32, WF32, bffn32, R832)
    pooled_exp = jnp.repeat(pooled32.reshape(HH, KK).T, DD, axis=1)

    hgat, y1 = _tc_gatout(hg.reshape(NC, NN, FD), invd, rs, E8, bg128)

    p1 = _sc_adjacency(y1, src, dst, zz128)
    tx1, y2 = _tc_cheb_step(p1.reshape(NC, NN, FD), rs, None)
    p2 = _sc_adjacency(y2, src, dst, zz128)
    tx2, y3 = _tc_cheb_step(p2.reshape(NC, NN, FD), rs, hgat)
    p3 = _sc_adjacency(y3, src, dst, zz128)

    return _tc_final(p3.reshape(NC, NN, FD), rs, tx1, tx2, hgat,
                     pooled_exp, BD, BDfl, bc128, bfl128, h)


# speedup vs baseline: 86.3785x; 1.4313x over previous
import functools

import jax
import jax.numpy as jnp
from jax import lax
from jax.experimental import pallas as pl
from jax.experimental.pallas import tpu as pltpu
from jax.experimental.pallas import tpu_sc as plsc

NN = 10000
EE = 320000
FD = 128
GD = 256
HH = 8
DD = 16
KK = 4

NC = 2
NS = 16
NW = NC * NS
EPT = EE // NW
C = 40
NCHUNK = EPT // C
CG = 80
NCHUNKG = EPT // CG
CA = 80
NCHUNKA = EPT // CA
NP_ = 10240
ROWS = NP_ // NS

f32 = jnp.float32
i32 = jnp.int32


def _mesh():
    return plsc.VectorSubcoreMesh(core_axis_name="c", subcore_axis_name="s")


R624 = 624
TAIL0 = 15 * R624 + R624


def _sc_gat(G, B, src, dst, m16):
    @functools.partial(
        pl.kernel,
        out_type=[
            jax.ShapeDtypeStruct((EE, 16), f32),
            jax.ShapeDtypeStruct((EE, FD), f32),
        ],
        mesh=_mesh(),
        scratch_types=[
            pltpu.VMEM((2, CG), i32),
            pltpu.VMEM((2, CG), i32),
            pltpu.VMEM((2 * CG, GD), f32),
            pltpu.VMEM((2 * CG, FD), f32),
            pltpu.VMEM((2 * CG, FD), f32),
            pltpu.VMEM((2 * CG, 16), f32),
            pltpu.VMEM((16,), f32),
            pltpu.SemaphoreType.DMA,
            pltpu.SemaphoreType.DMA,
            pltpu.SemaphoreType.DMA,
        ],
    )
    def k(G_h, B_h, src_h, dst_h, m_h, ee_h, msg_h,
          si, di, ga, bb, msg, eer, mv, gsem, ssem, esem):
        cid = lax.axis_index("c")
        sid = lax.axis_index("s")
        wid = sid * NC + cid
        pltpu.sync_copy(m_h, mv)
        mvec = mv[...]
        base0 = wid * EPT

        def idx_and_gather(i, p):
            base = base0 + i * CG
            pltpu.sync_copy(src_h.at[pl.ds(base, CG)], si.at[p])
            pltpu.sync_copy(dst_h.at[pl.ds(base, CG)], di.at[p])
            pltpu.async_copy(G_h.at[si.at[p]], ga.at[pl.ds(p * CG, CG)], gsem)
            pltpu.async_copy(B_h.at[di.at[p]], bb.at[pl.ds(p * CG, CG)], gsem)

        idx_and_gather(0, 0)

        def chunk(i, carry):
            p = lax.rem(i, 2)
            q = 1 - p

            @pl.when(i >= 1)
            def _():
                baseq = base0 + (i - 1) * CG
                pltpu.make_async_copy(
                    msg.at[pl.ds(q * CG, CG)], msg_h.at[pl.ds(baseq, CG)],
                    ssem).wait()
                pltpu.make_async_copy(
                    eer.at[pl.ds(q * CG, CG)], ee_h.at[pl.ds(baseq, CG)],
                    esem).wait()

            @pl.when(i < NCHUNKG - 1)
            def _():
                idx_and_gather(i + 1, q)

            pltpu.make_async_copy(
                G_h.at[si.at[p]], ga.at[pl.ds(p * CG, CG)], gsem).wait()
            pltpu.make_async_copy(
                B_h.at[di.at[p]], bb.at[pl.ds(p * CG, CG)], gsem).wait()

            def inner(c, carry2):
                r = p * CG + c
                v = ga[r, pl.ds(FD, 16)] + bb[r, pl.ds(0, 16)]
                v = jnp.where(v > 0.0, v, 0.2 * v)
                v = jnp.exp(v - mvec)
                eer[r, :] = v
                for hh in range(HH):
                    sl = pl.ds(hh * DD, DD)
                    msg[r, sl] = ga[r, sl] * v[hh]
                return carry2

            lax.fori_loop(0, CG, inner, 0, unroll=2)
            base = base0 + i * CG
            pltpu.async_copy(eer.at[pl.ds(p * CG, CG)],
                             ee_h.at[pl.ds(base, CG)], esem)
            pltpu.async_copy(msg.at[pl.ds(p * CG, CG)],
                             msg_h.at[pl.ds(base, CG)], ssem)
            return carry

        lax.fori_loop(0, NCHUNKG, chunk, 0)
        pf = lax.rem(NCHUNKG - 1, 2)
        basef = base0 + (NCHUNKG - 1) * CG
        pltpu.make_async_copy(
            msg.at[pl.ds(pf * CG, CG)], msg_h.at[pl.ds(basef, CG)],
            ssem).wait()
        pltpu.make_async_copy(
            eer.at[pl.ds(pf * CG, CG)], ee_h.at[pl.ds(basef, CG)],
            esem).wait()

    return k(G, B, src, dst, m16)


def _sc_accum(rows, dst, zz128):
    @functools.partial(
        pl.kernel,
        out_type=jax.ShapeDtypeStruct((NC * NN, FD), f32),
        mesh=_mesh(),
        scratch_types=[
            pltpu.VMEM((2, CA), i32),
            pltpu.VMEM((2 * CA, FD), f32),
            pltpu.VMEM_SHARED((NN, FD), f32),
            pltpu.SemaphoreType.DMA,
            pltpu.SemaphoreType.DMA,
        ],
    )
    def k(rows_h, dst_h, zz_h, out_h, di, rr, accS, gsem, ssem):
        cid = lax.axis_index("c")
        sid = lax.axis_index("s")
        wid = sid * NC + cid
        pltpu.sync_copy(zz_h.at[pl.ds(sid * R624, R624)],
                        accS.at[pl.ds(sid * R624, R624)])

        @pl.when(sid == NS - 1)
        def _():
            pltpu.sync_copy(zz_h.at[pl.ds(TAIL0, NN - TAIL0)],
                            accS.at[pl.ds(TAIL0, NN - TAIL0)])

        plsc.subcore_barrier()
        base0 = wid * EPT

        def idx_and_read(i, p):
            base = base0 + i * CA
            pltpu.sync_copy(dst_h.at[pl.ds(base, CA)], di.at[p])
            pltpu.async_copy(rows_h.at[pl.ds(base, CA)],
                             rr.at[pl.ds(p * CA, CA)], gsem)

        idx_and_read(0, 0)

        def chunk(i, carry):
            p = lax.rem(i, 2)
            q = 1 - p

            @pl.when(i >= 1)
            def _():
                pltpu.make_async_copy(
                    rr.at[pl.ds(q * CA, CA)], accS.at[di.at[q]], ssem).wait()

            @pl.when(i < NCHUNKA - 1)
            def _():
                idx_and_read(i + 1, q)

            base = base0 + i * CA
            pltpu.make_async_copy(
                rows_h.at[pl.ds(base, CA)], rr.at[pl.ds(p * CA, CA)],
                gsem).wait()
            pltpu.async_copy(rr.at[pl.ds(p * CA, CA)], accS.at[di.at[p]],
                             ssem, add=True)
            return carry

        lax.fori_loop(0, NCHUNKA, chunk, 0)
        pf = lax.rem(NCHUNKA - 1, 2)
        pltpu.make_async_copy(
            rr.at[pl.ds(pf * CA, CA)], accS.at[di.at[pf]], ssem).wait()
        plsc.subcore_barrier()
        pltpu.sync_copy(accS.at[pl.ds(sid * R624, R624)],
                        out_h.at[pl.ds(cid * NN + sid * R624, R624)])

        @pl.when(sid == NS - 1)
        def _():
            pltpu.sync_copy(accS.at[pl.ds(TAIL0, NN - TAIL0)],
                            out_h.at[pl.ds(cid * NN + TAIL0, NN - TAIL0)])

    return k(rows, dst, zz128)


def _sc_esum(ee, dst, zz128):
    @functools.partial(
        pl.kernel,
        out_type=jax.ShapeDtypeStruct((NC * NN, FD), f32),
        mesh=_mesh(),
        scratch_types=[
            pltpu.VMEM((C,), i32),
            pltpu.VMEM((C,), i32),
            pltpu.VMEM((C, 16), f32),
            pltpu.VMEM((C, 16), f32),
            pltpu.VMEM((C, FD), f32),
            pltpu.VMEM((C, FD), f32),
            pltpu.VMEM_SHARED((NN, FD), f32),
            pltpu.SemaphoreType.DMA,
            pltpu.SemaphoreType.DMA,
            pltpu.SemaphoreType.DMA,
            pltpu.SemaphoreType.DMA,
        ],
    )
    def k(ee_h, dst_h, zz128_h, acc_h, di0, di1, ev0, ev1, w0, w1, accS,
          g0, g1, s0, s1):
        cid = lax.axis_index("c")
        sid = lax.axis_index("s")
        wid = sid * NC + cid
        pltpu.sync_copy(zz128_h.at[pl.ds(sid * R624, R624)],
                        accS.at[pl.ds(sid * R624, R624)])

        @pl.when(sid == NS - 1)
        def _():
            pltpu.sync_copy(zz128_h.at[pl.ds(TAIL0, NN - TAIL0)],
                            accS.at[pl.ds(TAIL0, NN - TAIL0)])

        pltpu.sync_copy(zz128_h.at[pl.ds(0, C)], w0)
        pltpu.sync_copy(zz128_h.at[pl.ds(0, C)], w1)
        plsc.subcore_barrier()
        base0 = wid * EPT

        def expand(ev, w):
            def inner(c, carry2):
                w[c, pl.ds(0, 16)] = ev[c, :]
                return carry2

            lax.fori_loop(0, C, inner, 0, unroll=8)

        def pair(j, carry):
            a0 = base0 + (2 * j) * C
            a1 = a0 + C
            pltpu.sync_copy(dst_h.at[pl.ds(a0, C)], di0)
            cg0 = pltpu.async_copy(ee_h.at[pl.ds(a0, C)], ev0, g0)
            pltpu.sync_copy(dst_h.at[pl.ds(a1, C)], di1)
            cg1 = pltpu.async_copy(ee_h.at[pl.ds(a1, C)], ev1, g1)
            cg0.wait()
            expand(ev0, w0)
            cs0 = pltpu.async_copy(w0, accS.at[di0], s0, add=True)
            cg1.wait()
            expand(ev1, w1)
            cs1 = pltpu.async_copy(w1, accS.at[di1], s1, add=True)
            cs0.wait()
            cs1.wait()
            return carry

        lax.fori_loop(0, NCHUNK // 2, pair, 0)
        plsc.subcore_barrier()
        pltpu.sync_copy(accS.at[pl.ds(sid * R624, R624)],
                        acc_h.at[pl.ds(cid * NN + sid * R624, R624)])

        @pl.when(sid == NS - 1)
        def _():
            pltpu.sync_copy(accS.at[pl.ds(TAIL0, NN - TAIL0)],
                            acc_h.at[pl.ds(cid * NN + TAIL0, NN - TAIL0)])

    return k(ee, dst, zz128)


def _sc_adjacency(x, src, dst, zz128):
    @functools.partial(
        pl.kernel,
        out_type=jax.ShapeDtypeStruct((NC * NN, FD), f32),
        mesh=_mesh(),
        scratch_types=[
            pltpu.VMEM((2, CA), i32),
            pltpu.VMEM((2, CA), i32),
            pltpu.VMEM((2 * CA, FD), f32),
            pltpu.VMEM_SHARED((NN, FD), f32),
            pltpu.SemaphoreType.DMA,
            pltpu.SemaphoreType.DMA,
        ],
    )
    def k(x_h, src_h, dst_h, zz_h, out_h, si, di, rr, accS, gsem, ssem):
        cid = lax.axis_index("c")
        sid = lax.axis_index("s")
        wid = sid * NC + cid
        pltpu.sync_copy(zz_h.at[pl.ds(sid * R624, R624)],
                        accS.at[pl.ds(sid * R624, R624)])

        @pl.when(sid == NS - 1)
        def _():
            pltpu.sync_copy(zz_h.at[pl.ds(TAIL0, NN - TAIL0)],
                            accS.at[pl.ds(TAIL0, NN - TAIL0)])

        plsc.subcore_barrier()
        base0 = wid * EPT

        def idx_and_gather(i, p):
            base = base0 + i * CA
            pltpu.sync_copy(src_h.at[pl.ds(base, CA)], si.at[p])
            pltpu.sync_copy(dst_h.at[pl.ds(base, CA)], di.at[p])
            pltpu.async_copy(x_h.at[si.at[p]], rr.at[pl.ds(p * CA, CA)], gsem)

        idx_and_gather(0, 0)

        def chunk(i, carry):
            p = lax.rem(i, 2)
            q = 1 - p

            @pl.when(i >= 1)
            def _():
                pltpu.make_async_copy(
                    rr.at[pl.ds(q * CA, CA)], accS.at[di.at[q]], ssem).wait()

            @pl.when(i < NCHUNKA - 1)
            def _():
                idx_and_gather(i + 1, q)

            pltpu.make_async_copy(
                x_h.at[si.at[p]], rr.at[pl.ds(p * CA, CA)], gsem).wait()
            pltpu.async_copy(rr.at[pl.ds(p * CA, CA)], accS.at[di.at[p]],
                             ssem, add=True)
            return carry

        lax.fori_loop(0, NCHUNKA, chunk, 0)
        pf = lax.rem(NCHUNKA - 1, 2)
        pltpu.make_async_copy(
            rr.at[pl.ds(pf * CA, CA)], accS.at[di.at[pf]], ssem).wait()
        plsc.subcore_barrier()
        pltpu.sync_copy(accS.at[pl.ds(sid * R624, R624)],
                        out_h.at[pl.ds(cid * NN + sid * R624, R624)])

        @pl.when(sid == NS - 1)
        def _():
            pltpu.sync_copy(accS.at[pl.ds(TAIL0, NN - TAIL0)],
                            out_h.at[pl.ds(cid * NN + TAIL0, NN - TAIL0)])

    return k(x, src, dst, zz128)


_NB = 5
_BLK = NN // _NB


def _tc_proj(h, W_gat, AL, AR):
    def body(h_ref, w_ref, al_ref, ar_ref, g_ref, b_ref, el_ref, er_ref):
        f = jnp.dot(h_ref[...], w_ref[...], preferred_element_type=f32)
        el = jnp.dot(f, al_ref[...], preferred_element_type=f32)
        er = jnp.dot(f, ar_ref[...], preferred_element_type=f32)
        z = jnp.zeros((_BLK, GD - FD - HH), f32)
        g_ref[...] = jnp.concatenate([f, el, z], axis=1)
        b_ref[...] = jnp.concatenate([er, jnp.zeros((_BLK, FD - HH), f32)], axis=1)
        el_ref[...] = el
        er_ref[...] = er

    return pl.pallas_call(
        body,
        grid=(_NB,),
        in_specs=[
            pl.BlockSpec((_BLK, FD), lambda i: (i, 0)),
            pl.BlockSpec((FD, FD), lambda i: (0, 0)),
            pl.BlockSpec((FD, HH), lambda i: (0, 0)),
            pl.BlockSpec((FD, HH), lambda i: (0, 0)),
        ],
        out_specs=[
            pl.BlockSpec((_BLK, GD), lambda i: (i, 0)),
            pl.BlockSpec((_BLK, FD), lambda i: (i, 0)),
            pl.BlockSpec((_BLK, HH), lambda i: (i, 0)),
            pl.BlockSpec((_BLK, HH), lambda i: (i, 0)),
        ],
        out_shape=[
            jax.ShapeDtypeStruct((NN, GD), f32),
            jax.ShapeDtypeStruct((NN, FD), f32),
            jax.ShapeDtypeStruct((NN, HH), f32),
            jax.ShapeDtypeStruct((NN, HH), f32),
        ],
    )(h, W_gat, AL, AR)


def _tc_maxm(el, er):
    def body(el_ref, er_ref, m_ref):
        m_ref[...] = jnp.full(
            (1, 1),
            jnp.maximum(jnp.max(el_ref[...]) + jnp.max(er_ref[...]), 0.0), f32)

    return pl.pallas_call(
        body,
        out_shape=jax.ShapeDtypeStruct((1, 1), f32),
    )(el, er)


def _tc_denom(acc2, W_mp, bmp32, WF32, bffn32, R832):
    def body(acc_ref, wmp_ref, bmp_ref, wf_ref, bffn_ref, r_ref,
             invd_ref, rs_ref, pooled_ref):
        acc = acc_ref[0] + acc_ref[1]
        esum = acc[:, :HH]
        invd = 1.0 / (esum + 1e-16)
        invd_ref[...] = invd
        cnt = acc[:, HH:HH + 1]
        deg = jnp.maximum(cnt, 1.0)
        rs_ref[...] = lax.rsqrt(deg)
        s = esum * invd
        s32 = jnp.dot(s, r_ref[...], preferred_element_type=f32)
        colsum = jnp.sum(wmp_ref[...], axis=0, keepdims=True)
        cs32 = jnp.concatenate([colsum] * HH, axis=1)
        xc = jnp.tanh(s32 * cs32 + bmp_ref[...])
        hg = jnp.mean(xc, axis=0, keepdims=True)
        pooled_ref[...] = (
            jnp.dot(hg, wf_ref[...], preferred_element_type=f32) + bffn_ref[...])

    return pl.pallas_call(
        body,
        out_shape=[
            jax.ShapeDtypeStruct((NN, HH), f32),
            jax.ShapeDtypeStruct((NN, 1), f32),
            jax.ShapeDtypeStruct((1, HH * KK), f32),
        ],
    )(acc2, W_mp, bmp32, WF32, bffn32, R832)


def _tc_gatout(hg2, invd, rs, E8, bg128):
    def body(hg_ref, invd_ref, rs_ref, e8_ref, bg_ref, hgat_ref, y_ref):
        invd128 = jnp.dot(invd_ref[...], e8_ref[...], preferred_element_type=f32)
        hgat = (hg_ref[0] + hg_ref[1]) * invd128 + bg_ref[...]
        hgat_ref[...] = hgat
        y_ref[...] = rs_ref[...] * hgat

    return pl.pallas_call(
        body,
        grid=(_NB,),
        in_specs=[
            pl.BlockSpec((2, _BLK, FD), lambda i: (0, i, 0)),
            pl.BlockSpec((_BLK, HH), lambda i: (i, 0)),
            pl.BlockSpec((_BLK, 1), lambda i: (i, 0)),
            pl.BlockSpec((HH, FD), lambda i: (0, 0)),
            pl.BlockSpec((1, FD), lambda i: (0, 0)),
        ],
        out_specs=[
            pl.BlockSpec((_BLK, FD), lambda i: (i, 0)),
            pl.BlockSpec((_BLK, FD), lambda i: (i, 0)),
        ],
        out_shape=[
            jax.ShapeDtypeStruct((NN, FD), f32),
            jax.ShapeDtypeStruct((NN, FD), f32),
        ],
    )(hg2, invd, rs, E8, bg128)


def _tc_cheb_step(p2, rs, prev):
    first = prev is None
    coef = -1.0 if first else -2.0

    def body(*refs):
        if first:
            p_ref, rs_ref, tx_ref, y_ref = refs
            tx = coef * rs_ref[...] * (p_ref[0] + p_ref[1])
        else:
            p_ref, rs_ref, prev_ref, tx_ref, y_ref = refs
            tx = coef * rs_ref[...] * (p_ref[0] + p_ref[1]) - prev_ref[...]
        tx_ref[...] = tx
        y_ref[...] = rs_ref[...] * tx

    in_specs = [
        pl.BlockSpec((2, _BLK, FD), lambda i: (0, i, 0)),
        pl.BlockSpec((_BLK, 1), lambda i: (i, 0)),
    ]
    args = [p2, rs]
    if not first:
        in_specs.append(pl.BlockSpec((_BLK, FD), lambda i: (i, 0)))
        args.append(prev)
    return pl.pallas_call(
        body,
        grid=(_NB,),
        in_specs=in_specs,
        out_specs=[
            pl.BlockSpec((_BLK, FD), lambda i: (i, 0)),
            pl.BlockSpec((_BLK, FD), lambda i: (i, 0)),
        ],
        out_shape=[
            jax.ShapeDtypeStruct((NN, FD), f32),
            jax.ShapeDtypeStruct((NN, FD), f32),
        ],
    )(*args)


def _tc_final(p2, rs, tx1, tx2, hgat, pooled_exp, BD, BDfl, bc128, bfl128, h_in):
    def body(p_ref, rs_ref, tx1_ref, tx2_ref, hgat_ref, pe_ref, bd_ref,
             bdfl_ref, bc_ref, bfl_ref, hin_ref, out_ref):
        tx3 = -2.0 * rs_ref[...] * (p_ref[0] + p_ref[1]) - tx1_ref[...]
        txs = (hgat_ref[...], tx1_ref[...], tx2_ref[...], tx3)
        acc = jnp.zeros((_BLK, FD), f32) + bc_ref[...]
        for k in range(KK):
            acc = acc + jnp.dot(pe_ref[k:k + 1, :] * txs[k], bd_ref[k],
                                preferred_element_type=f32)
        hf = jnp.dot(jnp.tanh(acc), bdfl_ref[...],
                     preferred_element_type=f32) + bfl_ref[...]
        hh = hgat_ref[...] + hf
        hh = jnp.where(hh > 0.0, hh, jnp.exp(jnp.minimum(hh, 0.0)) - 1.0)
        out_ref[...] = hin_ref[...] + hh

    return pl.pallas_call(
        body,
        grid=(_NB,),
        in_specs=[
            pl.BlockSpec((2, _BLK, FD), lambda i: (0, i, 0)),
            pl.BlockSpec((_BLK, 1), lambda i: (i, 0)),
            pl.BlockSpec((_BLK, FD), lambda i: (i, 0)),
            pl.BlockSpec((_BLK, FD), lambda i: (i, 0)),
            pl.BlockSpec((_BLK, FD), lambda i: (i, 0)),
            pl.BlockSpec((KK, FD), lambda i: (0, 0)),
            pl.BlockSpec((KK, FD, FD), lambda i: (0, 0, 0)),
            pl.BlockSpec((FD, FD), lambda i: (0, 0)),
            pl.BlockSpec((1, FD), lambda i: (0, 0)),
            pl.BlockSpec((1, FD), lambda i: (0, 0)),
            pl.BlockSpec((_BLK, FD), lambda i: (i, 0)),
        ],
        out_specs=[pl.BlockSpec((_BLK, FD), lambda i: (i, 0))],
        out_shape=[jax.ShapeDtypeStruct((NN, FD), f32)],
    )(p2, rs, tx1, tx2, hgat, pooled_exp, BD, BDfl, bc128, bfl128, h_in)[0]


def kernel(h, edge_index, W_gat, attn_l, attn_r, b_gat, W_mp, b_mp,
           W_cheb, b_cheb, W_ffn, b_ffn, W_fl, b_fl):
    src = edge_index[0].astype(i32)
    dst = edge_index[1].astype(i32)

    lanes = jnp.arange(FD)
    rowh = lanes // DD
    AL = jnp.zeros((FD, HH), f32).at[lanes, rowh].set(attn_l.reshape(-1))
    AR = jnp.zeros((FD, HH), f32).at[lanes, rowh].set(attn_r.reshape(-1))
    E8 = jnp.zeros((HH, FD), f32).at[rowh, lanes].set(1.0)
    R832 = jnp.zeros((HH, HH * KK), f32).at[
        jnp.arange(HH * KK) // KK, jnp.arange(HH * KK)].set(1.0)
    bmp32 = jnp.tile(b_mp, HH).reshape(1, HH * KK)
    WF32 = jnp.kron(jnp.eye(HH, dtype=f32), W_ffn)
    bffn32 = jnp.tile(b_ffn, HH).reshape(1, HH * KK)
    eye8 = jnp.eye(HH, dtype=f32)
    BD = jnp.stack([jnp.kron(eye8, W_cheb[k]) for k in range(KK)])
    BDfl = jnp.kron(eye8, W_fl)
    bg128 = b_gat.reshape(1, FD)
    bc128 = jnp.tile(b_cheb, HH).reshape(1, FD)
    bfl128 = jnp.tile(b_fl, HH).reshape(1, FD)
    zz128 = jnp.zeros((NP_, FD), f32)

    G, B128, el, er = _tc_proj(h, W_gat, AL, AR)
    m1 = _tc_maxm(el, er)
    m16 = jnp.concatenate(
        [jnp.broadcast_to(m1.reshape(1), (HH,)), jnp.zeros((HH,), f32)])

    ee, msg = _sc_gat(G, B128, src, dst, m16)
    hg = _sc_accum(msg, dst, zz128)
    acc = _sc_esum(ee, dst, zz128)

    invd, rs, pooled32 = _tc_denom(acc.reshape(NC, NN, FD)[:, :, :16],
                                   W_mp, bmp32, WF32, bffn32, R832)
    pooled_exp = jnp.repeat(pooled32.reshape(HH, KK).T, DD, axis=1)

    hgat, y1 = _tc_gatout(hg.reshape(NC, NN, FD), invd, rs, E8, bg128)

    p1 = _sc_adjacency(y1, src, dst, zz128)
    tx1, y2 = _tc_cheb_step(p1.reshape(NC, NN, FD), rs, None)
    p2 = _sc_adjacency(y2, src, dst, zz128)
    tx2, y3 = _tc_cheb_step(p2.reshape(NC, NN, FD), rs, hgat)
    p3 = _sc_adjacency(y3, src, dst, zz128)

    return _tc_final(p3.reshape(NC, NN, FD), rs, tx1, tx2, hgat,
                     pooled_exp, BD, BDfl, bc128, bfl128, h)


# speedup vs baseline: 86.3880x; 1.0001x over previous
import functools

import jax
import jax.numpy as jnp
from jax import lax
from jax.experimental import pallas as pl
from jax.experimental.pallas import tpu as pltpu
from jax.experimental.pallas import tpu_sc as plsc

NN = 10000
EE = 320000
FD = 128
GD = 256
HH = 8
DD = 16
KK = 4

NC = 2
NS = 16
NW = NC * NS
EPT = EE // NW
C = 40
NCHUNK = EPT // C
CG = 80
NCHUNKG = EPT // CG
CA = 80
NCHUNKA = EPT // CA
NP_ = 10240
ROWS = NP_ // NS

f32 = jnp.float32
i32 = jnp.int32


def _mesh():
    return plsc.VectorSubcoreMesh(core_axis_name="c", subcore_axis_name="s")


R624 = 624
TAIL0 = 15 * R624 + R624


def _sc_gat(G, B, src, dst, m16):
    @functools.partial(
        pl.kernel,
        out_type=[
            jax.ShapeDtypeStruct((EE, 16), f32),
            jax.ShapeDtypeStruct((EE, FD), f32),
        ],
        mesh=_mesh(),
        scratch_types=[
            pltpu.VMEM((2, CG), i32),
            pltpu.VMEM((2, CG), i32),
            pltpu.VMEM((2 * CG, GD), f32),
            pltpu.VMEM((2 * CG, FD), f32),
            pltpu.VMEM((2 * CG, FD), f32),
            pltpu.VMEM((2 * CG, 16), f32),
            pltpu.VMEM((16,), f32),
            pltpu.SemaphoreType.DMA,
            pltpu.SemaphoreType.DMA,
            pltpu.SemaphoreType.DMA,
        ],
    )
    def k(G_h, B_h, src_h, dst_h, m_h, ee_h, msg_h,
          si, di, ga, bb, msg, eer, mv, gsem, ssem, esem):
        cid = lax.axis_index("c")
        sid = lax.axis_index("s")
        wid = sid * NC + cid
        pltpu.sync_copy(m_h, mv)
        mvec = mv[...]
        base0 = wid * EPT

        def idx_and_gather(i, p):
            base = base0 + i * CG
            pltpu.sync_copy(src_h.at[pl.ds(base, CG)], si.at[p])
            pltpu.sync_copy(dst_h.at[pl.ds(base, CG)], di.at[p])
            pltpu.async_copy(G_h.at[si.at[p]], ga.at[pl.ds(p * CG, CG)], gsem)
            pltpu.async_copy(B_h.at[di.at[p]], bb.at[pl.ds(p * CG, CG)], gsem)

        idx_and_gather(0, 0)

        def chunk(i, carry):
            p = lax.rem(i, 2)
            q = 1 - p

            @pl.when(i >= 1)
            def _():
                baseq = base0 + (i - 1) * CG
                pltpu.make_async_copy(
                    msg.at[pl.ds(q * CG, CG)], msg_h.at[pl.ds(baseq, CG)],
                    ssem).wait()
                pltpu.make_async_copy(
                    eer.at[pl.ds(q * CG, CG)], ee_h.at[pl.ds(baseq, CG)],
                    esem).wait()

            @pl.when(i < NCHUNKG - 1)
            def _():
                idx_and_gather(i + 1, q)

            pltpu.make_async_copy(
                G_h.at[si.at[p]], ga.at[pl.ds(p * CG, CG)], gsem).wait()
            pltpu.make_async_copy(
                B_h.at[di.at[p]], bb.at[pl.ds(p * CG, CG)], gsem).wait()

            def inner(c, carry2):
                r = p * CG + c
                v = ga[r, pl.ds(FD, 16)] + bb[r, pl.ds(0, 16)]
                v = jnp.where(v > 0.0, v, 0.2 * v)
                v = jnp.exp(v - mvec)
                eer[r, :] = v
                for hh in range(HH):
                    sl = pl.ds(hh * DD, DD)
                    msg[r, sl] = ga[r, sl] * v[hh]
                return carry2

            lax.fori_loop(0, CG, inner, 0, unroll=8)
            base = base0 + i * CG
            pltpu.async_copy(eer.at[pl.ds(p * CG, CG)],
                             ee_h.at[pl.ds(base, CG)], esem)
            pltpu.async_copy(msg.at[pl.ds(p * CG, CG)],
                             msg_h.at[pl.ds(base, CG)], ssem)
            return carry

        lax.fori_loop(0, NCHUNKG, chunk, 0)
        pf = lax.rem(NCHUNKG - 1, 2)
        basef = base0 + (NCHUNKG - 1) * CG
        pltpu.make_async_copy(
            msg.at[pl.ds(pf * CG, CG)], msg_h.at[pl.ds(basef, CG)],
            ssem).wait()
        pltpu.make_async_copy(
            eer.at[pl.ds(pf * CG, CG)], ee_h.at[pl.ds(basef, CG)],
            esem).wait()

    return k(G, B, src, dst, m16)


def _sc_accum(rows, dst, zz128):
    @functools.partial(
        pl.kernel,
        out_type=jax.ShapeDtypeStruct((NC * NN, FD), f32),
        mesh=_mesh(),
        scratch_types=[
            pltpu.VMEM((2, CA), i32),
            pltpu.VMEM((2 * CA, FD), f32),
            pltpu.VMEM_SHARED((NN, FD), f32),
            pltpu.SemaphoreType.DMA,
            pltpu.SemaphoreType.DMA,
        ],
    )
    def k(rows_h, dst_h, zz_h, out_h, di, rr, accS, gsem, ssem):
        cid = lax.axis_index("c")
        sid = lax.axis_index("s")
        wid = sid * NC + cid
        pltpu.sync_copy(zz_h.at[pl.ds(sid * R624, R624)],
                        accS.at[pl.ds(sid * R624, R624)])

        @pl.when(sid == NS - 1)
        def _():
            pltpu.sync_copy(zz_h.at[pl.ds(TAIL0, NN - TAIL0)],
                            accS.at[pl.ds(TAIL0, NN - TAIL0)])

        plsc.subcore_barrier()
        base0 = wid * EPT

        def idx_and_read(i, p):
            base = base0 + i * CA
            pltpu.sync_copy(dst_h.at[pl.ds(base, CA)], di.at[p])
            pltpu.async_copy(rows_h.at[pl.ds(base, CA)],
                             rr.at[pl.ds(p * CA, CA)], gsem)

        idx_and_read(0, 0)

        def chunk(i, carry):
            p = lax.rem(i, 2)
            q = 1 - p

            @pl.when(i >= 1)
            def _():
                pltpu.make_async_copy(
                    rr.at[pl.ds(q * CA, CA)], accS.at[di.at[q]], ssem).wait()

            @pl.when(i < NCHUNKA - 1)
            def _():
                idx_and_read(i + 1, q)

            base = base0 + i * CA
            pltpu.make_async_copy(
                rows_h.at[pl.ds(base, CA)], rr.at[pl.ds(p * CA, CA)],
                gsem).wait()
            pltpu.async_copy(rr.at[pl.ds(p * CA, CA)], accS.at[di.at[p]],
                             ssem, add=True)
            return carry

        lax.fori_loop(0, NCHUNKA, chunk, 0)
        pf = lax.rem(NCHUNKA - 1, 2)
        pltpu.make_async_copy(
            rr.at[pl.ds(pf * CA, CA)], accS.at[di.at[pf]], ssem).wait()
        plsc.subcore_barrier()
        pltpu.sync_copy(accS.at[pl.ds(sid * R624, R624)],
                        out_h.at[pl.ds(cid * NN + sid * R624, R624)])

        @pl.when(sid == NS - 1)
        def _():
            pltpu.sync_copy(accS.at[pl.ds(TAIL0, NN - TAIL0)],
                            out_h.at[pl.ds(cid * NN + TAIL0, NN - TAIL0)])

    return k(rows, dst, zz128)


def _sc_esum(ee, dst, zz128):
    @functools.partial(
        pl.kernel,
        out_type=jax.ShapeDtypeStruct((NC * NN, FD), f32),
        mesh=_mesh(),
        scratch_types=[
            pltpu.VMEM((C,), i32),
            pltpu.VMEM((C,), i32),
            pltpu.VMEM((C, 16), f32),
            pltpu.VMEM((C, 16), f32),
            pltpu.VMEM((C, FD), f32),
            pltpu.VMEM((C, FD), f32),
            pltpu.VMEM_SHARED((NN, FD), f32),
            pltpu.SemaphoreType.DMA,
            pltpu.SemaphoreType.DMA,
            pltpu.SemaphoreType.DMA,
            pltpu.SemaphoreType.DMA,
        ],
    )
    def k(ee_h, dst_h, zz128_h, acc_h, di0, di1, ev0, ev1, w0, w1, accS,
          g0, g1, s0, s1):
        cid = lax.axis_index("c")
        sid = lax.axis_index("s")
        wid = sid * NC + cid
        pltpu.sync_copy(zz128_h.at[pl.ds(sid * R624, R624)],
                        accS.at[pl.ds(sid * R624, R624)])

        @pl.when(sid == NS - 1)
        def _():
            pltpu.sync_copy(zz128_h.at[pl.ds(TAIL0, NN - TAIL0)],
                            accS.at[pl.ds(TAIL0, NN - TAIL0)])

        pltpu.sync_copy(zz128_h.at[pl.ds(0, C)], w0)
        pltpu.sync_copy(zz128_h.at[pl.ds(0, C)], w1)
        plsc.subcore_barrier()
        base0 = wid * EPT

        def expand(ev, w):
            def inner(c, carry2):
                w[c, pl.ds(0, 16)] = ev[c, :]
                return carry2

            lax.fori_loop(0, C, inner, 0, unroll=8)

        def pair(j, carry):
            a0 = base0 + (2 * j) * C
            a1 = a0 + C
            pltpu.sync_copy(dst_h.at[pl.ds(a0, C)], di0)
            cg0 = pltpu.async_copy(ee_h.at[pl.ds(a0, C)], ev0, g0)
            pltpu.sync_copy(dst_h.at[pl.ds(a1, C)], di1)
            cg1 = pltpu.async_copy(ee_h.at[pl.ds(a1, C)], ev1, g1)
            cg0.wait()
            expand(ev0, w0)
            cs0 = pltpu.async_copy(w0, accS.at[di0], s0, add=True)
            cg1.wait()
            expand(ev1, w1)
            cs1 = pltpu.async_copy(w1, accS.at[di1], s1, add=True)
            cs0.wait()
            cs1.wait()
            return carry

        lax.fori_loop(0, NCHUNK // 2, pair, 0)
        plsc.subcore_barrier()
        pltpu.sync_copy(accS.at[pl.ds(sid * R624, R624)],
                        acc_h.at[pl.ds(cid * NN + sid * R624, R624)])

        @pl.when(sid == NS - 1)
        def _():
            pltpu.sync_copy(accS.at[pl.ds(TAIL0, NN - TAIL0)],
                            acc_h.at[pl.ds(cid * NN + TAIL0, NN - TAIL0)])

    return k(ee, dst, zz128)


def _sc_adjacency(x, src, dst, zz128):
    @functools.partial(
        pl.kernel,
        out_type=jax.ShapeDtypeStruct((NC * NN, FD), f32),
        mesh=_mesh(),
        scratch_types=[
            pltpu.VMEM((2, CA), i32),
            pltpu.VMEM((2, CA), i32),
            pltpu.VMEM((2 * CA, FD), f32),
            pltpu.VMEM_SHARED((NN, FD), f32),
            pltpu.SemaphoreType.DMA,
            pltpu.SemaphoreType.DMA,
        ],
    )
    def k(x_h, src_h, dst_h, zz_h, out_h, si, di, rr, accS, gsem, ssem):
        cid = lax.axis_index("c")
        sid = lax.axis_index("s")
        wid = sid * NC + cid
        pltpu.sync_copy(zz_h.at[pl.ds(sid * R624, R624)],
                        accS.at[pl.ds(sid * R624, R624)])

        @pl.when(sid == NS - 1)
        def _():
            pltpu.sync_copy(zz_h.at[pl.ds(TAIL0, NN - TAIL0)],
                            accS.at[pl.ds(TAIL0, NN - TAIL0)])

        plsc.subcore_barrier()
        base0 = wid * EPT

        def idx_and_gather(i, p):
            base = base0 + i * CA
            pltpu.sync_copy(src_h.at[pl.ds(base, CA)], si.at[p])
            pltpu.sync_copy(dst_h.at[pl.ds(base, CA)], di.at[p])
            pltpu.async_copy(x_h.at[si.at[p]], rr.at[pl.ds(p * CA, CA)], gsem)

        idx_and_gather(0, 0)

        def chunk(i, carry):
            p = lax.rem(i, 2)
            q = 1 - p

            @pl.when(i >= 1)
            def _():
                pltpu.make_async_copy(
                    rr.at[pl.ds(q * CA, CA)], accS.at[di.at[q]], ssem).wait()

            @pl.when(i < NCHUNKA - 1)
            def _():
                idx_and_gather(i + 1, q)

            pltpu.make_async_copy(
                x_h.at[si.at[p]], rr.at[pl.ds(p * CA, CA)], gsem).wait()
            pltpu.async_copy(rr.at[pl.ds(p * CA, CA)], accS.at[di.at[p]],
                             ssem, add=True)
            return carry

        lax.fori_loop(0, NCHUNKA, chunk, 0)
        pf = lax.rem(NCHUNKA - 1, 2)
        pltpu.make_async_copy(
            rr.at[pl.ds(pf * CA, CA)], accS.at[di.at[pf]], ssem).wait()
        plsc.subcore_barrier()
        pltpu.sync_copy(accS.at[pl.ds(sid * R624, R624)],
                        out_h.at[pl.ds(cid * NN + sid * R624, R624)])

        @pl.when(sid == NS - 1)
        def _():
            pltpu.sync_copy(accS.at[pl.ds(TAIL0, NN - TAIL0)],
                            out_h.at[pl.ds(cid * NN + TAIL0, NN - TAIL0)])

    return k(x, src, dst, zz128)


_NB = 5
_BLK = NN // _NB


def _tc_proj(h, W_gat, AL, AR):
    def body(h_ref, w_ref, al_ref, ar_ref, g_ref, b_ref, el_ref, er_ref):
        f = jnp.dot(h_ref[...], w_ref[...], preferred_element_type=f32)
        el = jnp.dot(f, al_ref[...], preferred_element_type=f32)
        er = jnp.dot(f, ar_ref[...], preferred_element_type=f32)
        z = jnp.zeros((_BLK, GD - FD - HH), f32)
        g_ref[...] = jnp.concatenate([f, el, z], axis=1)
        b_ref[...] = jnp.concatenate([er, jnp.zeros((_BLK, FD - HH), f32)], axis=1)
        el_ref[...] = el
        er_ref[...] = er

    return pl.pallas_call(
        body,
        grid=(_NB,),
        in_specs=[
            pl.BlockSpec((_BLK, FD), lambda i: (i, 0)),
            pl.BlockSpec((FD, FD), lambda i: (0, 0)),
            pl.BlockSpec((FD, HH), lambda i: (0, 0)),
            pl.BlockSpec((FD, HH), lambda i: (0, 0)),
        ],
        out_specs=[
            pl.BlockSpec((_BLK, GD), lambda i: (i, 0)),
            pl.BlockSpec((_BLK, FD), lambda i: (i, 0)),
            pl.BlockSpec((_BLK, HH), lambda i: (i, 0)),
            pl.BlockSpec((_BLK, HH), lambda i: (i, 0)),
        ],
        out_shape=[
            jax.ShapeDtypeStruct((NN, GD), f32),
            jax.ShapeDtypeStruct((NN, FD), f32),
            jax.ShapeDtypeStruct((NN, HH), f32),
            jax.ShapeDtypeStruct((NN, HH), f32),
        ],
    )(h, W_gat, AL, AR)


def _tc_maxm(el, er):
    def body(el_ref, er_ref, m_ref):
        m_ref[...] = jnp.full(
            (1, 1),
            jnp.maximum(jnp.max(el_ref[...]) + jnp.max(er_ref[...]), 0.0), f32)

    return pl.pallas_call(
        body,
        out_shape=jax.ShapeDtypeStruct((1, 1), f32),
    )(el, er)


def _tc_denom(acc2, W_mp, bmp32, WF32, bffn32, R832):
    def body(acc_ref, wmp_ref, bmp_ref, wf_ref, bffn_ref, r_ref,
             invd_ref, rs_ref, pooled_ref):
        acc = acc_ref[0] + acc_ref[1]
        esum = acc[:, :HH]
        invd = 1.0 / (esum + 1e-16)
        invd_ref[...] = invd
        cnt = acc[:, HH:HH + 1]
        deg = jnp.maximum(cnt, 1.0)
        rs_ref[...] = lax.rsqrt(deg)
        s = esum * invd
        s32 = jnp.dot(s, r_ref[...], preferred_element_type=f32)
        colsum = jnp.sum(wmp_ref[...], axis=0, keepdims=True)
        cs32 = jnp.concatenate([colsum] * HH, axis=1)
        xc = jnp.tanh(s32 * cs32 + bmp_ref[...])
        hg = jnp.mean(xc, axis=0, keepdims=True)
        pooled_ref[...] = (
            jnp.dot(hg, wf_ref[...], preferred_element_type=f32) + bffn_ref[...])

    return pl.pallas_call(
        body,
        out_shape=[
            jax.ShapeDtypeStruct((NN, HH), f32),
            jax.ShapeDtypeStruct((NN, 1), f32),
            jax.ShapeDtypeStruct((1, HH * KK), f32),
        ],
    )(acc2, W_mp, bmp32, WF32, bffn32, R832)


def _tc_gatout(hg2, invd, rs, E8, bg128):
    def body(hg_ref, invd_ref, rs_ref, e8_ref, bg_ref, hgat_ref, y_ref):
        invd128 = jnp.dot(invd_ref[...], e8_ref[...], preferred_element_type=f32)
        hgat = (hg_ref[0] + hg_ref[1]) * invd128 + bg_ref[...]
        hgat_ref[...] = hgat
        y_ref[...] = rs_ref[...] * hgat

    return pl.pallas_call(
        body,
        grid=(_NB,),
        in_specs=[
            pl.BlockSpec((2, _BLK, FD), lambda i: (0, i, 0)),
            pl.BlockSpec((_BLK, HH), lambda i: (i, 0)),
            pl.BlockSpec((_BLK, 1), lambda i: (i, 0)),
            pl.BlockSpec((HH, FD), lambda i: (0, 0)),
            pl.BlockSpec((1, FD), lambda i: (0, 0)),
        ],
        out_specs=[
            pl.BlockSpec((_BLK, FD), lambda i: (i, 0)),
            pl.BlockSpec((_BLK, FD), lambda i: (i, 0)),
        ],
        out_shape=[
            jax.ShapeDtypeStruct((NN, FD), f32),
            jax.ShapeDtypeStruct((NN, FD), f32),
        ],
    )(hg2, invd, rs, E8, bg128)


def _tc_cheb_step(p2, rs, prev):
    first = prev is None
    coef = -1.0 if first else -2.0

    def body(*refs):
        if first:
            p_ref, rs_ref, tx_ref, y_ref = refs
            tx = coef * rs_ref[...] * (p_ref[0] + p_ref[1])
        else:
            p_ref, rs_ref, prev_ref, tx_ref, y_ref = refs
            tx = coef * rs_ref[...] * (p_ref[0] + p_ref[1]) - prev_ref[...]
        tx_ref[...] = tx
        y_ref[...] = rs_ref[...] * tx

    in_specs = [
        pl.BlockSpec((2, _BLK, FD), lambda i: (0, i, 0)),
        pl.BlockSpec((_BLK, 1), lambda i: (i, 0)),
    ]
    args = [p2, rs]
    if not first:
        in_specs.append(pl.BlockSpec((_BLK, FD), lambda i: (i, 0)))
        args.append(prev)
    return pl.pallas_call(
        body,
        grid=(_NB,),
        in_specs=in_specs,
        out_specs=[
            pl.BlockSpec((_BLK, FD), lambda i: (i, 0)),
            pl.BlockSpec((_BLK, FD), lambda i: (i, 0)),
        ],
        out_shape=[
            jax.ShapeDtypeStruct((NN, FD), f32),
            jax.ShapeDtypeStruct((NN, FD), f32),
        ],
    )(*args)


def _tc_final(p2, rs, tx1, tx2, hgat, pooled_exp, BD, BDfl, bc128, bfl128, h_in):
    def body(p_ref, rs_ref, tx1_ref, tx2_ref, hgat_ref, pe_ref, bd_ref,
             bdfl_ref, bc_ref, bfl_ref, hin_ref, out_ref):
        tx3 = -2.0 * rs_ref[...] * (p_ref[0] + p_ref[1]) - tx1_ref[...]
        txs = (hgat_ref[...], tx1_ref[...], tx2_ref[...], tx3)
        acc = jnp.zeros((_BLK, FD), f32) + bc_ref[...]
        for k in range(KK):
            acc = acc + jnp.dot(pe_ref[k:k + 1, :] * txs[k], bd_ref[k],
                                preferred_element_type=f32)
        hf = jnp.dot(jnp.tanh(acc), bdfl_ref[...],
                     preferred_element_type=f32) + bfl_ref[...]
        hh = hgat_ref[...] + hf
        hh = jnp.where(hh > 0.0, hh, jnp.exp(jnp.minimum(hh, 0.0)) - 1.0)
        out_ref[...] = hin_ref[...] + hh

    return pl.pallas_call(
        body,
        grid=(_NB,),
        in_specs=[
            pl.BlockSpec((2, _BLK, FD), lambda i: (0, i, 0)),
            pl.BlockSpec((_BLK, 1), lambda i: (i, 0)),
            pl.BlockSpec((_BLK, FD), lambda i: (i, 0)),
            pl.BlockSpec((_BLK, FD), lambda i: (i, 0)),
            pl.BlockSpec((_BLK, FD), lambda i: (i, 0)),
            pl.BlockSpec((KK, FD), lambda i: (0, 0)),
            pl.BlockSpec((KK, FD, FD), lambda i: (0, 0, 0)),
            pl.BlockSpec((FD, FD), lambda i: (0, 0)),
            pl.BlockSpec((1, FD), lambda i: (0, 0)),
            pl.BlockSpec((1, FD), lambda i: (0, 0)),
            pl.BlockSpec((_BLK, FD), lambda i: (i, 0)),
        ],
        out_specs=[pl.BlockSpec((_BLK, FD), lambda i: (i, 0))],
        out_shape=[jax.ShapeDtypeStruct((NN, FD), f32)],
    )(p2, rs, tx1, tx2, hgat, pooled_exp, BD, BDfl, bc128, bfl128, h_in)[0]


def kernel(h, edge_index, W_gat, attn_l, attn_r, b_gat, W_mp, b_mp,
           W_cheb, b_cheb, W_ffn, b_ffn, W_fl, b_fl):
    src = edge_index[0].astype(i32)
    dst = edge_index[1].astype(i32)

    lanes = jnp.arange(FD)
    rowh = lanes // DD
    AL = jnp.zeros((FD, HH), f32).at[lanes, rowh].set(attn_l.reshape(-1))
    AR = jnp.zeros((FD, HH), f32).at[lanes, rowh].set(attn_r.reshape(-1))
    E8 = jnp.zeros((HH, FD), f32).at[rowh, lanes].set(1.0)
    R832 = jnp.zeros((HH, HH * KK), f32).at[
        jnp.arange(HH * KK) // KK, jnp.arange(HH * KK)].set(1.0)
    bmp32 = jnp.tile(b_mp, HH).reshape(1, HH * KK)
    WF32 = jnp.kron(jnp.eye(HH, dtype=f32), W_ffn)
    bffn32 = jnp.tile(b_ffn, HH).reshape(1, HH * KK)
    eye8 = jnp.eye(HH, dtype=f32)
    BD = jnp.stack([jnp.kron(eye8, W_cheb[k]) for k in range(KK)])
    BDfl = jnp.kron(eye8, W_fl)
    bg128 = b_gat.reshape(1, FD)
    bc128 = jnp.tile(b_cheb, HH).reshape(1, FD)
    bfl128 = jnp.tile(b_fl, HH).reshape(1, FD)
    zz128 = jnp.zeros((NP_, FD), f32)

    G, B128, el, er = _tc_proj(h, W_gat, AL, AR)
    m1 = _tc_maxm(el, er)
    m16 = jnp.concatenate(
        [jnp.broadcast_to(m1.reshape(1), (HH,)), jnp.zeros((HH,), f32)])

    ee, msg = _sc_gat(G, B128, src, dst, m16)
    hg = _sc_accum(msg, dst, zz128)
    acc = _sc_esum(ee, dst, zz128)

    invd, rs, pooled32 = _tc_denom(acc.reshape(NC, NN, FD)[:, :, :16],
                                   W_mp, bmp32, WF32, bffn32, R832)
    pooled_exp = jnp.repeat(pooled32.reshape(HH, KK).T, DD, axis=1)

    hgat, y1 = _tc_gatout(hg.reshape(NC, NN, FD), invd, rs, E8, bg128)

    p1 = _sc_adjacency(y1, src, dst, zz128)
    tx1, y2 = _tc_cheb_step(p1.reshape(NC, NN, FD), rs, None)
    p2 = _sc_adjacency(y2, src, dst, zz128)
    tx2, y3 = _tc_cheb_step(p2.reshape(NC, NN, FD), rs, hgat)
    p3 = _sc_adjacency(y3, src, dst, zz128)

    return _tc_final(p3.reshape(NC, NN, FD), rs, tx1, tx2, hgat,
                     pooled_exp, BD, BDfl, bc128, bfl128, h)
